# baseline (speedup 1.0000x reference)
import jax
import jax.numpy as jnp
from jax import lax
from jax.experimental import pallas as pl
from jax.experimental.pallas import tpu as pltpu

_B, _N, _C = 8, 2048, 131
_NPT = 512


def _fps_kernel(x2d_ref, xt_ref, out_ref):
    iota = lax.broadcasted_iota(jnp.int32, (1, _N), 1)

    a_sq = []
    for b in range(_B):
        xb = xt_ref[b]
        a_sq.append(jnp.sum(xb * xb, axis=0, keepdims=True))

    def _extract(row, idx):
        return jnp.max(jnp.where(iota == idx, row, -jnp.inf))

    def body(i, carry):
        fs, sqfs, dists = carry
        new_fs, new_sqfs, new_dists = [], [], []
        for b in range(_B):
            out_ref[i, b] = fs[b]
            row = x2d_ref[pl.ds(b * _N + fs[b], 1), :]
            corr = lax.dot_general(
                row, xt_ref[b],
                dimension_numbers=(((1,), (0,)), ((), ())),
                preferred_element_type=jnp.float32,
            )
            d = (sqfs[b] + a_sq[b]) - 2.0 * corr
            nd = jnp.minimum(dists[b], d)
            m = jnp.max(nd)
            nf = jnp.min(jnp.where(nd == m, iota, _N)).astype(jnp.int32)
            new_fs.append(nf)
            new_sqfs.append(_extract(a_sq[b], nf))
            new_dists.append(nd)
        return tuple(new_fs), tuple(new_sqfs), tuple(new_dists)

    fs0 = tuple(jnp.int32(0) for _ in range(_B))
    sqf0 = tuple(_extract(a_sq[b], jnp.int32(0)) for b in range(_B))
    dists0 = tuple(jnp.full((1, _N), 1e10, jnp.float32) for _ in range(_B))
    lax.fori_loop(0, _NPT, body, (fs0, sqf0, dists0))


def kernel(points_xyz, features):
    feats_t = jnp.transpose(features, (0, 2, 1))
    xcat = jnp.concatenate([points_xyz, feats_t], axis=2)
    x2d = xcat.reshape(_B * _N, _C)
    xt = jnp.concatenate(
        [jnp.transpose(points_xyz, (0, 2, 1)), features], axis=1)
    out = pl.pallas_call(
        _fps_kernel,
        out_shape=jax.ShapeDtypeStruct((_NPT, _B), jnp.int32),
        in_specs=[
            pl.BlockSpec(memory_space=pltpu.VMEM),
            pl.BlockSpec(memory_space=pltpu.VMEM),
        ],
        out_specs=pl.BlockSpec(memory_space=pltpu.SMEM),
    )(x2d, xt)
    return jnp.transpose(out, (1, 0))


# speedup vs baseline: 1.8857x; 1.8857x over previous
import jax
import jax.numpy as jnp
from jax import lax
from jax.experimental import pallas as pl
from jax.experimental.pallas import tpu as pltpu

_B, _N, _C = 8, 2048, 131
_CP = 256
_NPT = 512


def _fps_kernel(x2d_ref, xts_ref, out_ref, bd_ref, asq_ref):
    iota = lax.broadcasted_iota(jnp.int32, (_B, _N), 1)
    iota8 = lax.broadcasted_iota(jnp.int32, (_B, 1), 0)

    bd_ref[...] = jnp.zeros((_B, _B * _CP), jnp.float32)
    for b in range(_B):
        xb = xts_ref[b * _CP:b * _CP + _C, :]
        asq_ref[b:b + 1, :] = jnp.sum(xb * xb, axis=0, keepdims=True)
    a_sq = asq_ref[...]

    def _scal(vec, b):
        return jnp.max(jnp.where(iota8 == b, vec, -1))

    def _row_extract(mat, idx_vec, fill):
        return jnp.max(jnp.where(iota == idx_vec, mat, fill), axis=1,
                       keepdims=True)

    def body(i, carry):
        fs, sqf, dists = carry
        for b in range(_B):
            out_ref[i, b] = fs[b]
            row = x2d_ref[pl.ds(b * _N + fs[b], 1), :]
            bd_ref[b:b + 1, b * _CP:b * _CP + _C] = row
        corr = lax.dot_general(
            bd_ref[...], xts_ref[...],
            dimension_numbers=(((1,), (0,)), ((), ())),
            preferred_element_type=jnp.float32,
        )
        d = (sqf + a_sq) - 2.0 * corr
        nd = jnp.minimum(dists, d)
        m = jnp.max(nd, axis=1, keepdims=True)
        nf = jnp.min(jnp.where(nd == m, iota, _N), axis=1,
                     keepdims=True).astype(jnp.int32)
        nsqf = _row_extract(a_sq, nf, -jnp.inf)
        nfs = tuple(_scal(nf, b) for b in range(_B))
        return nfs, nsqf, nd

    fs0 = tuple(jnp.int32(0) for _ in range(_B))
    sqf0 = _row_extract(a_sq, jnp.zeros((_B, 1), jnp.int32), -jnp.inf)
    dists0 = jnp.full((_B, _N), 1e10, jnp.float32)
    lax.fori_loop(0, _NPT, body, (fs0, sqf0, dists0))


def kernel(points_xyz, features):
    feats_t = jnp.transpose(features, (0, 2, 1))
    xcat = jnp.concatenate([points_xyz, feats_t], axis=2)
    x2d = xcat.reshape(_B * _N, _C)
    xt = jnp.concatenate(
        [jnp.transpose(points_xyz, (0, 2, 1)), features], axis=1)
    xts = jnp.pad(xt, ((0, 0), (0, _CP - _C), (0, 0))).reshape(_B * _CP, _N)
    out = pl.pallas_call(
        _fps_kernel,
        out_shape=jax.ShapeDtypeStruct((_NPT, _B), jnp.int32),
        in_specs=[
            pl.BlockSpec(memory_space=pltpu.VMEM),
            pl.BlockSpec(memory_space=pltpu.VMEM),
        ],
        out_specs=pl.BlockSpec(memory_space=pltpu.SMEM),
        scratch_shapes=[
            pltpu.VMEM((_B, _B * _CP), jnp.float32),
            pltpu.VMEM((_B, _N), jnp.float32),
        ],
    )(x2d, xts)
    return jnp.transpose(out, (1, 0))


# speedup vs baseline: 2.6644x; 1.4130x over previous
import jax
import jax.numpy as jnp
from jax import lax
from jax.experimental import pallas as pl
from jax.experimental.pallas import tpu as pltpu

_B, _N, _C = 8, 2048, 131
_NPT = 512


def _fps_kernel(x2d_ref, xt_ref, out_ref, corr_ref, asq_ref):
    iota = lax.broadcasted_iota(jnp.int32, (_B, _N), 1)
    iota8 = lax.broadcasted_iota(jnp.int32, (_B, 1), 0)

    for b in range(_B):
        xb = xt_ref[b]
        asq_ref[b:b + 1, :] = jnp.sum(xb * xb, axis=0, keepdims=True)
    a_sq = asq_ref[...]

    def _scal(vec, b):
        return jnp.max(jnp.where(iota8 == b, vec, -1))

    def _row_extract(mat, idx_vec, fill):
        return jnp.max(jnp.where(iota == idx_vec, mat, fill), axis=1,
                       keepdims=True)

    def body(i, carry):
        fs, sqf, dists = carry
        for b in range(_B):
            out_ref[i, b] = fs[b]
            row = x2d_ref[pl.ds(b * _N + fs[b], 1), :]
            corr_ref[b:b + 1, :] = lax.dot_general(
                row, xt_ref[b],
                dimension_numbers=(((1,), (0,)), ((), ())),
                preferred_element_type=jnp.float32,
            )
        corr = corr_ref[...]
        d = (sqf + a_sq) - 2.0 * corr
        nd = jnp.minimum(dists, d)
        m = jnp.max(nd, axis=1, keepdims=True)
        nf = jnp.min(jnp.where(nd == m, iota, _N), axis=1,
                     keepdims=True).astype(jnp.int32)
        nsqf = _row_extract(a_sq, nf, -jnp.inf)
        nfs = tuple(_scal(nf, b) for b in range(_B))
        return nfs, nsqf, nd

    fs0 = tuple(jnp.int32(0) for _ in range(_B))
    sqf0 = _row_extract(a_sq, jnp.zeros((_B, 1), jnp.int32), -jnp.inf)
    dists0 = jnp.full((_B, _N), 1e10, jnp.float32)
    lax.fori_loop(0, _NPT, body, (fs0, sqf0, dists0))


def kernel(points_xyz, features):
    feats_t = jnp.transpose(features, (0, 2, 1))
    xcat = jnp.concatenate([points_xyz, feats_t], axis=2)
    x2d = xcat.reshape(_B * _N, _C)
    xt = jnp.concatenate(
        [jnp.transpose(points_xyz, (0, 2, 1)), features], axis=1)
    out = pl.pallas_call(
        _fps_kernel,
        out_shape=jax.ShapeDtypeStruct((_NPT, _B), jnp.int32),
        in_specs=[
            pl.BlockSpec(memory_space=pltpu.VMEM),
            pl.BlockSpec(memory_space=pltpu.VMEM),
        ],
        out_specs=pl.BlockSpec(memory_space=pltpu.SMEM),
        scratch_shapes=[
            pltpu.VMEM((_B, _N), jnp.float32),
            pltpu.VMEM((_B, _N), jnp.float32),
        ],
    )(x2d, xt)
    return jnp.transpose(out, (1, 0))
